# baseline (speedup 1.0000x reference)
import functools
import math

import jax
import jax.numpy as jnp
from jax import lax
from jax.experimental import pallas as pl
from jax.experimental.pallas import tpu as pltpu
from jax.experimental.pallas import tpu_sc as plsc

D = 64
EPS = 1e-6
SQRT_D = math.sqrt(D)

NC = 2
NS = 16
NW = NC * NS

ROWS_PER_STAGE = 2
L = 16


def _fused_kernel(table, x, pos8t2, gamma, beta):
    B, S = x.shape
    b_per_w = B // NW
    n_stages = b_per_w // ROWS_PER_STAGE
    tok = ROWS_PER_STAGE * S
    n_groups = tok // L
    mesh = plsc.VectorSubcoreMesh(core_axis_name="c", subcore_axis_name="s")

    @functools.partial(
        pl.kernel,
        out_type=jax.ShapeDtypeStruct((B, S, D), jnp.float32),
        mesh=mesh,
        scratch_types=[
            pltpu.VMEM((b_per_w, S), jnp.int32),
            pltpu.VMEM((2, tok, D), jnp.float32),
            pltpu.VMEM((S, D), jnp.float32),
            pltpu.VMEM((2, D), jnp.float32),
            pltpu.SemaphoreType.DMA,
            pltpu.SemaphoreType.DMA,
        ],
        compiler_params=pltpu.CompilerParams(
            use_tc_tiling_on_sc=False, needs_layout_passes=False
        ),
    )
    def body(table_h, x_h, pos_h, gam_h, bet_h, out_h, idx_v, slab_v, pos_v,
             gb_v, gsem, osem):
        wid = lax.axis_index("s") * NC + lax.axis_index("c")
        b0 = wid * b_per_w
        pltpu.sync_copy(x_h.at[pl.ds(b0, b_per_w)], idx_v)
        pltpu.sync_copy(pos_h, pos_v)
        pltpu.sync_copy(gam_h, gb_v.at[0])
        pltpu.sync_copy(bet_h, gb_v.at[1])

        def fire_gathers(stage, slab):
            for r in range(ROWS_PER_STAGE):
                row = stage * ROWS_PER_STAGE + r
                pltpu.async_copy(
                    table_h.at[idx_v.at[row, pl.ds(0, 128)]],
                    slab_v.at[slab, pl.ds(r * S, 128)], gsem)
                pltpu.async_copy(
                    table_h.at[idx_v.at[row, pl.ds(128, S - 128)]],
                    slab_v.at[slab, pl.ds(r * S + 128, S - 128)], gsem)

        def wait_gathers(slab):
            pltpu.make_async_copy(
                table_h.at[pl.ds(0, tok)], slab_v.at[slab], gsem).wait()

        def fire_outs(stage, slab):
            bb = b0 + stage * ROWS_PER_STAGE
            for r in range(ROWS_PER_STAGE):
                pltpu.async_copy(
                    slab_v.at[slab, pl.ds(r * S, S)], out_h.at[bb + r], osem)

        def wait_outs(slab):
            for r in range(ROWS_PER_STAGE):
                pltpu.make_async_copy(
                    slab_v.at[slab, pl.ds(r * S, S)], out_h.at[b0], osem).wait()

        inv_d = 1.0 / D
        inv_dm1 = 1.0 / (D - 1)
        eps8 = EPS / SQRT_D
        nv = D // L

        def compute(slab):
            gvec = [gb_v[0, pl.ds(L * k, L)] for k in range(nv)]
            bvec = [gb_v[1, pl.ds(L * k, L)] for k in range(nv)]

            def s_body(s, carry):
                p = [pos_v[s, pl.ds(L * k, L)] for k in range(nv)]
                for r in range(ROWS_PER_STAGE):
                    t = r * S + s
                    h = [
                        slab_v[slab, t, pl.ds(L * k, L)] + p[k]
                        for k in range(nv)
                    ]
                    hs = (h[0] + h[1]) + (h[2] + h[3])
                    hq = (h[0] * h[0] + h[1] * h[1]) + (
                        h[2] * h[2] + h[3] * h[3])
                    sm = jnp.sum(hs)
                    sq = jnp.sum(hq)
                    mean = sm * inv_d
                    var = jnp.maximum((sq - sm * mean) * inv_dm1, 1e-30)
                    bits = lax.bitcast_convert_type(var, jnp.int32)
                    u = lax.bitcast_convert_type(
                        jnp.int32(0x5F3759DF) - (bits >> 1), jnp.float32)
                    for _ in range(3):
                        u = u * (1.5 - 0.5 * var * u * u)
                    rr = u - eps8 * (u * u)
                    mr = mean * rr
                    for k in range(nv):
                        o = (h[k] * rr - mr) * gvec[k] + bvec[k]
                        slab_v[slab, t, pl.ds(L * k, L)] = o
                return carry

            lax.fori_loop(0, S, s_body, 0)

        fire_gathers(0, 0)

        def stage_pair(j, carry):
            for slab in range(2):
                stg = 2 * j + slab
                wait_gathers(slab)

                @pl.when(stg >= 1)
                def _():
                    wait_outs(1 - slab)

                @pl.when(stg + 1 < n_stages)
                def _():
                    fire_gathers(stg + 1, 1 - slab)

                compute(slab)
                fire_outs(stg, slab)
            return carry

        lax.fori_loop(0, n_stages // 2, stage_pair, 0)
        wait_outs(1)

    return body(table, x, pos8t2, gamma, beta)


def kernel(x, table, gamma, beta, pos_embed):
    b, s = x.shape
    xi = x.astype(jnp.int32)
    pos = lax.slice(pos_embed, (0, 1, 0), (1, s + 1, D))[0]
    pos8 = pos * (1.0 / SQRT_D)
    return _fused_kernel(table, xi, pos8, gamma, beta)


# speedup vs baseline: 1.1628x; 1.1628x over previous
import functools
import math

import jax
import jax.numpy as jnp
from jax import lax
from jax.experimental import pallas as pl
from jax.experimental.pallas import tpu as pltpu
from jax.experimental.pallas import tpu_sc as plsc

D = 64
EPS = 1e-6
SQRT_D = math.sqrt(D)

NC = 2
NS = 16
NW = NC * NS

ROWS_PER_STAGE = 2
L = 16


def _fused_kernel(table, x, pos8t2, gamma, beta):
    B, S = x.shape
    b_per_w = B // NW
    n_stages = b_per_w // ROWS_PER_STAGE
    tok = ROWS_PER_STAGE * S
    n_groups = tok // L
    mesh = plsc.VectorSubcoreMesh(core_axis_name="c", subcore_axis_name="s")

    @functools.partial(
        pl.kernel,
        out_type=jax.ShapeDtypeStruct((B, S, D), jnp.float32),
        mesh=mesh,
        scratch_types=[
            pltpu.VMEM((b_per_w, S), jnp.int32),
            pltpu.VMEM((2, tok, D), jnp.float32),
            pltpu.VMEM((S, D), jnp.float32),
            pltpu.VMEM((2, D), jnp.float32),
            pltpu.SemaphoreType.DMA,
            pltpu.SemaphoreType.DMA,
        ],
        compiler_params=pltpu.CompilerParams(
            use_tc_tiling_on_sc=False, needs_layout_passes=False
        ),
    )
    def body(table_h, x_h, pos_h, gam_h, bet_h, out_h, idx_v, slab_v, pos_v,
             gb_v, gsem, osem):
        wid = lax.axis_index("s") * NC + lax.axis_index("c")
        b0 = wid * b_per_w
        pltpu.sync_copy(x_h.at[pl.ds(b0, b_per_w)], idx_v)
        pltpu.sync_copy(pos_h, pos_v)
        pltpu.sync_copy(gam_h, gb_v.at[0])
        pltpu.sync_copy(bet_h, gb_v.at[1])

        def fire_gathers(stage, slab):
            for r in range(ROWS_PER_STAGE):
                row = stage * ROWS_PER_STAGE + r
                pltpu.async_copy(
                    table_h.at[idx_v.at[row, pl.ds(0, 128)]],
                    slab_v.at[slab, pl.ds(r * S, 128)], gsem)
                pltpu.async_copy(
                    table_h.at[idx_v.at[row, pl.ds(128, S - 128)]],
                    slab_v.at[slab, pl.ds(r * S + 128, S - 128)], gsem)

        def wait_gathers(slab):
            pltpu.make_async_copy(
                table_h.at[pl.ds(0, tok)], slab_v.at[slab], gsem).wait()

        def fire_outs(stage, slab):
            bb = b0 + stage * ROWS_PER_STAGE
            for r in range(ROWS_PER_STAGE):
                pltpu.async_copy(
                    slab_v.at[slab, pl.ds(r * S, S)], out_h.at[bb + r], osem)

        def wait_outs(slab):
            for r in range(ROWS_PER_STAGE):
                pltpu.make_async_copy(
                    slab_v.at[slab, pl.ds(r * S, S)], out_h.at[b0], osem).wait()

        inv_d = 1.0 / D
        inv_dm1 = 1.0 / (D - 1)
        nv = D // L

        def compute(slab):
            gvec = [gb_v[0, pl.ds(L * k, L)] for k in range(nv)]
            bvec = [gb_v[1, pl.ds(L * k, L)] for k in range(nv)]

            def s_body(si, carry):
                toks = []
                for half in range(2):
                    s = si * 2 + half
                    p = [pos_v[s, pl.ds(L * k, L)] for k in range(nv)]
                    for r in range(ROWS_PER_STAGE):
                        t = r * S + s
                        h = [
                            slab_v[slab, t, pl.ds(L * k, L)] + p[k]
                            for k in range(nv)
                        ]
                        toks.append((t, h))
                stats = []
                for t, h in toks:
                    hs = (h[0] + h[1]) + (h[2] + h[3])
                    hq = (h[0] * h[0] + h[1] * h[1]) + (
                        h[2] * h[2] + h[3] * h[3])
                    stats.append((jnp.sum(hs), jnp.sum(hq)))
                for (t, h), (sm, sq) in zip(toks, stats):
                    mean = sm * inv_d
                    var = jnp.maximum((sq - sm * mean) * inv_dm1, 1e-30)
                    bits = lax.bitcast_convert_type(var, jnp.int32)
                    u = lax.bitcast_convert_type(
                        jnp.int32(0x5F3759DF) - (bits >> 1), jnp.float32)
                    for _ in range(2):
                        u = u * (1.5 - 0.5 * var * u * u)
                    mr = mean * u
                    for k in range(nv):
                        o = (h[k] * u - mr) * gvec[k] + bvec[k]
                        slab_v[slab, t, pl.ds(L * k, L)] = o
                return carry

            lax.fori_loop(0, S // 2, s_body, 0)

        fire_gathers(0, 0)

        def stage_pair(j, carry):
            for slab in range(2):
                stg = 2 * j + slab
                wait_gathers(slab)

                @pl.when(stg >= 1)
                def _():
                    wait_outs(1 - slab)

                @pl.when(stg + 1 < n_stages)
                def _():
                    fire_gathers(stg + 1, 1 - slab)

                compute(slab)
                fire_outs(stg, slab)
            return carry

        lax.fori_loop(0, n_stages // 2, stage_pair, 0)
        wait_outs(1)

    return body(table, x, pos8t2, gamma, beta)


def kernel(x, table, gamma, beta, pos_embed):
    b, s = x.shape
    xi = x.astype(jnp.int32)
    pos = lax.slice(pos_embed, (0, 1, 0), (1, s + 1, D))[0]
    pos8 = pos * (1.0 / SQRT_D)
    return _fused_kernel(table, xi, pos8, gamma, beta)
